# baseline (speedup 1.0000x reference)
import jax
import jax.numpy as jnp
from jax.experimental import pallas as pl
from jax.experimental.pallas import tpu as pltpu

RMIN, RMAX = 60, 68
IN_DIM, EMB = 128, 26
NPOS = RMAX - RMIN
NCARD = 52
TABLE = NPOS * NCARD
AW = IN_DIM + TABLE


def _body(x_ref, w_ref, o_ref, a_s):
    e = pl.program_id(0)
    b = x_ref.shape[0]

    @pl.when(e == 0)
    def _():
        a_s[:, :IN_DIM] = x_ref[...].astype(jnp.bfloat16)
        xs = x_ref[:, RMIN:RMAX]
        jm = jax.lax.broadcasted_iota(jnp.int32, (NPOS, TABLE), 1) // NCARD
        rj = jax.lax.broadcasted_iota(jnp.int32, (NPOS, TABLE), 0)
        rep = (jm == rj).astype(jnp.bfloat16)
        xs_rep = jnp.dot(xs.astype(jnp.bfloat16), rep,
                         preferred_element_type=jnp.float32)
        cm = (jax.lax.broadcasted_iota(jnp.int32, (b, TABLE), 1)
              % NCARD).astype(jnp.float32)
        a_s[:, IN_DIM:] = (xs_rep == cm).astype(jnp.bfloat16)

    o_ref[0] = jnp.dot(a_s[...], w_ref[0],
                       preferred_element_type=jnp.float32)


@jax.jit
def kernel(x, card_buffer):
    b = x.shape[0]
    i_idx = jnp.arange(IN_DIM)
    dense_mask = (i_idx[:, None] == i_idx[None, :]) & (
        (i_idx[:, None] < RMIN) | (i_idx[:, None] >= RMAX))
    diag = jnp.broadcast_to(dense_mask[None], (EMB, IN_DIM, IN_DIM))
    cbt = card_buffer.transpose(2, 0, 1).reshape(EMB, TABLE)
    pos_one = (i_idx[None, :] == (RMIN + jnp.arange(TABLE)[:, None] // NCARD))
    strip = cbt[:, :, None] * pos_one[None, :, :]
    w_all = jnp.concatenate([diag, strip], axis=1).astype(jnp.bfloat16)

    out_t = pl.pallas_call(
        _body,
        grid=(EMB,),
        in_specs=[
            pl.BlockSpec((b, IN_DIM), lambda e: (0, 0)),
            pl.BlockSpec((1, AW, IN_DIM), lambda e: (e, 0, 0)),
        ],
        out_specs=pl.BlockSpec((1, b, IN_DIM), lambda e: (e, 0, 0)),
        out_shape=jax.ShapeDtypeStruct((EMB, b, IN_DIM), jnp.float32),
        scratch_shapes=[pltpu.VMEM((b, AW), jnp.bfloat16)],
    )(x, w_all)
    return jnp.transpose(out_t, (1, 2, 0))


# speedup vs baseline: 1.1874x; 1.1874x over previous
import jax
import jax.numpy as jnp
from jax.experimental import pallas as pl
from jax.experimental.pallas import tpu as pltpu

RMIN, RMAX = 60, 68
IN_DIM, EMB = 128, 26
NPOS = RMAX - RMIN
NCARD = 52
TABLE = NPOS * NCARD


def _body(x_ref, w_ref, o_ref, ohm_s):
    e = pl.program_id(0)
    b = x_ref.shape[0]

    @pl.when(e == 0)
    def _():
        xs = x_ref[:, RMIN:RMAX]
        jm = jax.lax.broadcasted_iota(jnp.int32, (NPOS, TABLE), 1) // NCARD
        rj = jax.lax.broadcasted_iota(jnp.int32, (NPOS, TABLE), 0)
        rep = (jm == rj).astype(jnp.bfloat16)
        xs_rep = jnp.dot(xs.astype(jnp.bfloat16), rep,
                         preferred_element_type=jnp.float32)
        cm = (jax.lax.broadcasted_iota(jnp.int32, (b, TABLE), 1)
              % NCARD).astype(jnp.float32)
        ohm_s[...] = (xs_rep == cm).astype(jnp.bfloat16)

    g = jnp.dot(ohm_s[...], w_ref[0], preferred_element_type=jnp.float32)
    col = jax.lax.broadcasted_iota(jnp.int32, (b, IN_DIM), 1)
    mask = ((col < RMIN) | (col >= RMAX)).astype(jnp.float32)
    o_ref[0] = x_ref[...] * mask + g


@jax.jit
def kernel(x, card_buffer):
    b = x.shape[0]
    cbt = card_buffer.transpose(2, 0, 1).reshape(EMB, TABLE)
    pos_one = (jnp.arange(IN_DIM)[None, :]
               == (RMIN + jnp.arange(TABLE)[:, None] // NCARD))
    w_all = (cbt[:, :, None] * pos_one[None, :, :]).astype(jnp.bfloat16)

    out_t = pl.pallas_call(
        _body,
        grid=(EMB,),
        in_specs=[
            pl.BlockSpec((b, IN_DIM), lambda e: (0, 0)),
            pl.BlockSpec((1, TABLE, IN_DIM), lambda e: (e, 0, 0)),
        ],
        out_specs=pl.BlockSpec((1, b, IN_DIM), lambda e: (e, 0, 0)),
        out_shape=jax.ShapeDtypeStruct((EMB, b, IN_DIM), jnp.float32),
        scratch_shapes=[pltpu.VMEM((b, TABLE), jnp.bfloat16)],
    )(x, w_all)
    return jnp.transpose(out_t, (1, 2, 0))
